# initial kernel scaffold (unmeasured)
import jax
import jax.numpy as jnp
from jax import lax
from jax.experimental import pallas as pl
from jax.experimental.pallas import tpu as pltpu

N_DEV = 8
E_LOC = 4
CAP = 25
C = 32


def kernel(x, router_W, route_idx, expert_W):
    n_tok, d_model = x.shape
    d_out = expert_W.shape[2]
    rows = n_tok // N_DEV

    route_t = route_idx.reshape(1, n_tok)

    def body(x_ref, ri_ref, rit_ref, ew_ref, out_ref,
             partial_ref, comm_ref, send_sems, recv_sems):
        me = lax.axis_index("i")
        right = lax.rem(me + 1, N_DEV)

        ri = ri_ref[:, :]
        rit = rit_ref[:, :]
        k_col = lax.broadcasted_iota(jnp.int32, (n_tok, E_LOC), 1)
        ind = (ri == (k_col + me * E_LOC)).astype(jnp.float32)
        k_row = lax.broadcasted_iota(jnp.int32, (E_LOC, n_tok), 0)
        ind_t = (rit == (k_row + me * E_LOC)).astype(jnp.float32)

        row_i = lax.broadcasted_iota(jnp.int32, (n_tok, n_tok), 0)
        col_i = lax.broadcasted_iota(jnp.int32, (n_tok, n_tok), 1)
        L = (row_i > col_i).astype(jnp.float32)
        U = (row_i < col_i).astype(jnp.float32)
        rank = jnp.dot(L, ind, preferred_element_type=jnp.float32)
        rank = rank.astype(jnp.int32)
        rank_t = jnp.dot(ind_t, U, preferred_element_type=jnp.float32)
        rank_t = rank_t.astype(jnp.int32)

        x_all = x_ref[:, :]
        c_row = lax.broadcasted_iota(jnp.int32, (C, n_tok), 0)
        c_col = lax.broadcasted_iota(jnp.int32, (n_tok, C), 1)
        acc = jnp.zeros((n_tok, d_out), jnp.float32)
        for k in range(E_LOC):
            Dk = (((rank_t[k:k + 1, :] == c_row) & (c_row < CAP)).astype(jnp.float32)
                  * ind_t[k:k + 1, :])
            Xk = jnp.dot(Dk, x_all, preferred_element_type=jnp.float32)
            Yk = jnp.dot(Xk, ew_ref[k], preferred_element_type=jnp.float32)
            DkT = (((rank[:, k:k + 1] == c_col) & (c_col < CAP)).astype(jnp.float32)
                   * ind[:, k:k + 1])
            acc = acc + jnp.dot(DkT, Yk, preferred_element_type=jnp.float32)
        partial_ref[:, :] = acc

        start0 = lax.rem(me - 1 + N_DEV, N_DEV) * rows
        comm_ref[0, :, :] = partial_ref[pl.ds(start0, rows), :]
        for h in range(N_DEV - 1):
            rdma = pltpu.make_async_remote_copy(
                src_ref=comm_ref.at[h],
                dst_ref=comm_ref.at[h + 1],
                send_sem=send_sems.at[h],
                recv_sem=recv_sems.at[h],
                device_id=(right,),
                device_id_type=pl.DeviceIdType.MESH,
            )
            rdma.start()
            rdma.wait()
            c_start = lax.rem(me - h - 2 + 2 * N_DEV, N_DEV) * rows
            comm_ref[h + 1, :, :] += partial_ref[pl.ds(c_start, rows), :]
        out_ref[:, :] = comm_ref[N_DEV - 1, :, :]

    return pl.pallas_call(
        body,
        out_shape=jax.ShapeDtypeStruct((rows, d_out), jnp.float32),
        in_specs=[pl.BlockSpec(memory_space=pltpu.VMEM)] * 4,
        out_specs=pl.BlockSpec(memory_space=pltpu.VMEM),
        scratch_shapes=[
            pltpu.VMEM((n_tok, d_out), jnp.float32),
            pltpu.VMEM((N_DEV, rows, d_out), jnp.float32),
            pltpu.SemaphoreType.DMA((N_DEV - 1,)),
            pltpu.SemaphoreType.DMA((N_DEV - 1,)),
        ],
        compiler_params=pltpu.CompilerParams(collective_id=0),
    )(x, route_idx, route_t, expert_W)


# baseline (device time: 71359 ns/iter reference)
import jax
import jax.numpy as jnp
from jax import lax
from jax.experimental import pallas as pl
from jax.experimental.pallas import tpu as pltpu

N_DEV = 8
E_LOC = 4
CAP = 25
C = 32


def kernel(x, router_W, route_idx, expert_W):
    n_tok, d_model = x.shape
    d_out = expert_W.shape[2]
    rows = n_tok // N_DEV

    route_t = route_idx.reshape(1, n_tok)

    def body(x_ref, ri_ref, rit_ref, ew_ref, out_ref,
             partial_ref, comm_ref, send_sems, recv_sems):
        me = lax.axis_index("i")
        right = lax.rem(me + 1, N_DEV)

        ri = ri_ref[:, :]
        rit = rit_ref[:, :]
        k_col = lax.broadcasted_iota(jnp.int32, (n_tok, E_LOC), 1)
        ind = (ri == (k_col + me * E_LOC)).astype(jnp.float32)
        k_row = lax.broadcasted_iota(jnp.int32, (E_LOC, n_tok), 0)
        ind_t = (rit == (k_row + me * E_LOC)).astype(jnp.float32)

        row_i = lax.broadcasted_iota(jnp.int32, (n_tok, n_tok), 0)
        col_i = lax.broadcasted_iota(jnp.int32, (n_tok, n_tok), 1)
        L = (row_i > col_i).astype(jnp.float32)
        U = (row_i < col_i).astype(jnp.float32)
        rank = jnp.dot(L, ind, preferred_element_type=jnp.float32)
        rank = rank.astype(jnp.int32)
        rank_t = jnp.dot(ind_t, U, preferred_element_type=jnp.float32)
        rank_t = rank_t.astype(jnp.int32)

        x_all = x_ref[:, :]
        c_row = lax.broadcasted_iota(jnp.int32, (C, n_tok), 0)
        c_col = lax.broadcasted_iota(jnp.int32, (n_tok, C), 1)
        acc = jnp.zeros((n_tok, d_out), jnp.float32)
        for k in range(E_LOC):
            Dk = (((rank_t[k:k + 1, :] == c_row) & (c_row < CAP)).astype(jnp.float32)
                  * ind_t[k:k + 1, :])
            Xk = jnp.dot(Dk, x_all, preferred_element_type=jnp.float32)
            Yk = jnp.dot(Xk, ew_ref[k], preferred_element_type=jnp.float32)
            DkT = (((rank[:, k:k + 1] == c_col) & (c_col < CAP)).astype(jnp.float32)
                   * ind[:, k:k + 1])
            acc = acc + jnp.dot(DkT, Yk, preferred_element_type=jnp.float32)
        partial_ref[:, :] = acc

        start0 = lax.rem(me - 1 + N_DEV, N_DEV) * rows
        comm_ref[0, :, :] = partial_ref[pl.ds(start0, rows), :]
        for h in range(N_DEV - 1):
            rdma = pltpu.make_async_remote_copy(
                src_ref=comm_ref.at[h],
                dst_ref=comm_ref.at[h + 1],
                send_sem=send_sems.at[h],
                recv_sem=recv_sems.at[h],
                device_id=(right,),
                device_id_type=pl.DeviceIdType.MESH,
            )
            rdma.start()
            rdma.wait()
            c_start = lax.rem(me - h - 2 + 2 * N_DEV, N_DEV) * rows
            comm_ref[h + 1, :, :] += partial_ref[pl.ds(c_start, rows), :]
        out_ref[:, :] = comm_ref[N_DEV - 1, :, :]

    return pl.pallas_call(
        body,
        out_shape=jax.ShapeDtypeStruct((rows, d_out), jnp.float32),
        in_specs=[pl.BlockSpec(memory_space=pltpu.VMEM)] * 4,
        out_specs=pl.BlockSpec(memory_space=pltpu.VMEM),
        scratch_shapes=[
            pltpu.VMEM((n_tok, d_out), jnp.float32),
            pltpu.VMEM((N_DEV, rows, d_out), jnp.float32),
            pltpu.SemaphoreType.DMA((N_DEV - 1,)),
            pltpu.SemaphoreType.DMA((N_DEV - 1,)),
        ],
    )(x, route_idx, route_t, expert_W)


# device time: 32661 ns/iter; 2.1848x vs baseline; 2.1848x over previous
import jax
import jax.numpy as jnp
from jax import lax
from jax.experimental import pallas as pl
from jax.experimental.pallas import tpu as pltpu

N_DEV = 8
E_LOC = 4
CAP = 25
C = 32


def kernel(x, router_W, route_idx, expert_W):
    n_tok, d_model = x.shape
    d_out = expert_W.shape[2]
    rows = n_tok // N_DEV

    route_t = route_idx.reshape(1, n_tok)

    def body(x_ref, ri_ref, rit_ref, ew_ref, out_ref,
             rank_ref, ind_ref, send_ref, recv_ref, send_sems, recv_sems):
        me = lax.axis_index("i")

        ri = ri_ref[:, :]
        rit = rit_ref[:, :]
        k_col = lax.broadcasted_iota(jnp.int32, (n_tok, E_LOC), 1)
        ind = (ri == (k_col + me * E_LOC)).astype(jnp.float32)
        k_row = lax.broadcasted_iota(jnp.int32, (E_LOC, n_tok), 0)
        ind_t = (rit == (k_row + me * E_LOC)).astype(jnp.float32)

        row_i = lax.broadcasted_iota(jnp.int32, (n_tok, n_tok), 0)
        col_i = lax.broadcasted_iota(jnp.int32, (n_tok, n_tok), 1)
        L = (row_i > col_i).astype(jnp.float32)
        U = (row_i < col_i).astype(jnp.float32)
        rank = jnp.dot(L, ind, preferred_element_type=jnp.float32)
        rank_ref[:, :] = rank.astype(jnp.int32)
        ind_ref[:, :] = ind
        rank_t = jnp.dot(ind_t, U, preferred_element_type=jnp.float32)
        rank_t = rank_t.astype(jnp.int32)

        c_row = lax.broadcasted_iota(jnp.int32, (C, n_tok), 0)
        D_all = jnp.concatenate([
            ((rank_t[k:k + 1, :] == c_row) & (c_row < CAP)).astype(jnp.float32)
            * ind_t[k:k + 1, :]
            for k in range(E_LOC)
        ], axis=0)
        X_all = jnp.dot(D_all, x_ref[:, :],
                        preferred_element_type=jnp.float32)
        Y_all = jnp.concatenate([
            jnp.dot(X_all[k * C:(k + 1) * C], ew_ref[k],
                    preferred_element_type=jnp.float32)
            for k in range(E_LOC)
        ], axis=0)

        c_col = lax.broadcasted_iota(jnp.int32, (rows, C), 1)

        def dest_chunk(j):
            rj = rank_ref[pl.ds(j * rows, rows), :]
            ij = ind_ref[pl.ds(j * rows, rows), :]
            DjT = jnp.concatenate([
                ((rj[:, k:k + 1] == c_col) & (c_col < CAP)).astype(jnp.float32)
                * ij[:, k:k + 1]
                for k in range(E_LOC)
            ], axis=1)
            return jnp.dot(DjT, Y_all,
                           preferred_element_type=jnp.float32)

        rdmas = []
        for off in range(1, N_DEV):
            j = lax.rem(me + off, N_DEV)
            send_ref[off] = dest_chunk(j).astype(jnp.bfloat16)
            rdma = pltpu.make_async_remote_copy(
                src_ref=send_ref.at[off],
                dst_ref=recv_ref.at[off],
                send_sem=send_sems.at[off],
                recv_sem=recv_sems.at[off],
                device_id=(j,),
                device_id_type=pl.DeviceIdType.MESH,
            )
            rdma.start()
            rdmas.append(rdma)

        recv_ref[0] = dest_chunk(me).astype(jnp.bfloat16)

        for rdma in rdmas:
            rdma.wait_recv()
        for rdma in rdmas:
            rdma.wait_send()

        out_ref[:, :] = jnp.sum(recv_ref[:, :, :].astype(jnp.float32), axis=0)

    return pl.pallas_call(
        body,
        out_shape=jax.ShapeDtypeStruct((rows, d_out), jnp.float32),
        in_specs=[pl.BlockSpec(memory_space=pltpu.VMEM)] * 4,
        out_specs=pl.BlockSpec(memory_space=pltpu.VMEM),
        scratch_shapes=[
            pltpu.VMEM((n_tok, E_LOC), jnp.int32),
            pltpu.VMEM((n_tok, E_LOC), jnp.float32),
            pltpu.VMEM((N_DEV, rows, d_out), jnp.bfloat16),
            pltpu.VMEM((N_DEV, rows, d_out), jnp.bfloat16),
            pltpu.SemaphoreType.DMA((N_DEV,)),
            pltpu.SemaphoreType.DMA((N_DEV,)),
        ],
    )(x, route_idx, route_t, expert_W)
